# baseline (speedup 1.0000x reference)
import functools

import jax
import jax.numpy as jnp
from jax import lax
from jax.experimental import pallas as pl
from jax.experimental.pallas import tpu as pltpu
from jax.experimental.pallas import tpu_sc as plsc

_C = 128
_B = 14


def _prep_body(h_ref, norm_ref, o_ref):
    o_ref[...] = h_ref[...] * norm_ref[...]


def _main_body(h_ref, parts_ref, norm_ref, w1h_ref, w1a_ref, w2_ref, o_ref):
    agg = (parts_ref[0] + parts_ref[1]) * norm_ref[...]
    x = jnp.dot(h_ref[...], w1h_ref[...], preferred_element_type=jnp.float32)
    x = x + jnp.dot(agg, w1a_ref[...], preferred_element_type=jnp.float32)
    x = jnp.maximum(x, 0.0)
    s = jnp.maximum(jnp.sum(x * x, axis=1, keepdims=True), 1e-12)
    x = x * lax.rsqrt(s)
    y = jnp.dot(x, w2_ref[...], preferred_element_type=jnp.float32)
    o_ref[...] = jnp.maximum(y, 0.0)


@functools.partial(jax.jit, static_argnames=("nw", "nc", "ks", "rows"))
def _sc_scatter(normh, packed_w, *, nw, nc, ks, rows):
    d = normh.shape[1]
    ns = nw // nc
    k = max(ks)
    zch = rows // (ns * _C)
    mask = jnp.int32((1 << _B) - 1)

    @functools.partial(
        pl.kernel,
        out_type=jax.ShapeDtypeStruct((nc, rows, d), jnp.float32),
        mesh=plsc.VectorSubcoreMesh(core_axis_name="c", subcore_axis_name="s"),
        scratch_types=[
            pltpu.VMEM((k, _C), jnp.int32),
            pltpu.VMEM((_C,), jnp.int32),
            pltpu.VMEM((_C,), jnp.int32),
            pltpu.VMEM((_C,), jnp.int32),
            pltpu.VMEM((_C, d), jnp.float32),
            pltpu.VMEM((_C, d), jnp.float32),
            pltpu.VMEM_SHARED((rows, d), jnp.float32),
            pltpu.SemaphoreType.DMA,
            pltpu.SemaphoreType.DMA,
        ],
    )
    def kfn(normh_hbm, packed_hbm, out_hbm, pk_v, s0_v, s1_v, dst_v,
            buf0, buf1, agg, sem0, sem1):
        cid = lax.axis_index("c")
        sid = lax.axis_index("s")
        wid = sid * nc + cid
        kc = jnp.where(cid == 0, ks[0], ks[1])

        pltpu.sync_copy(packed_hbm.at[wid], pk_v)

        zero16 = jnp.zeros((16,), jnp.float32)

        def zrow(r, carry):
            for j in range(d // 16):
                buf0[r, pl.ds(j * 16, 16)] = zero16
            return carry

        lax.fori_loop(0, _C, zrow, 0)
        for j in range(zch):
            pltpu.sync_copy(buf0, agg.at[pl.ds((sid * zch + j) * _C, _C)])
        plsc.subcore_barrier()

        def unpack(row, idx_ref, shift):
            pk_row = pk_v.at[row]
            for j in range(_C // 16):
                p = pk_row[pl.ds(j * 16, 16)]
                if shift:
                    p = lax.shift_right_logical(p, _B)
                idx_ref[pl.ds(j * 16, 16)] = lax.bitwise_and(p, mask)

        def gather(idx_ref, buf, sem):
            return pltpu.async_copy(normh_hbm.at[idx_ref], buf, sem)

        def wait(idx_ref, buf, sem):
            pltpu.make_async_copy(normh_hbm.at[idx_ref], buf, sem).wait()

        unpack(0, s0_v, False)
        gather(s0_v, buf0, sem0)
        unpack(1, s1_v, False)
        gather(s1_v, buf1, sem1)

        def body(i, carry):
            c = 2 * i
            unpack(c, dst_v, True)
            wait(s0_v, buf0, sem0)
            pltpu.sync_copy(buf0, agg.at[dst_v], add=True)

            @pl.when(c + 2 < kc)
            def _():
                unpack(c + 2, s0_v, False)
                gather(s0_v, buf0, sem0)

            unpack(c + 1, dst_v, True)
            wait(s1_v, buf1, sem1)
            pltpu.sync_copy(buf1, agg.at[dst_v], add=True)

            @pl.when(c + 3 < kc)
            def _():
                unpack(c + 3, s1_v, False)
                gather(s1_v, buf1, sem1)

            return carry

        lax.fori_loop(0, kc // 2, body, 0)
        plsc.subcore_barrier()

        for j in range(zch):
            r0 = (sid * zch + j) * _C
            pltpu.sync_copy(agg.at[pl.ds(r0, _C)], out_hbm.at[cid, pl.ds(r0, _C)])

    return kfn(normh, packed_w)


def kernel(h, edge_index, norm, W1, W2):
    n, d = h.shape
    e = edge_index.shape[1]
    info = plsc.get_sparse_core_info()
    nc, ns = info.num_cores, info.num_subcores
    nw = nc * ns

    bn = 400
    grid = n // bn
    normh = pl.pallas_call(
        _prep_body,
        grid=(grid,),
        in_specs=[
            pl.BlockSpec((bn, d), lambda i: (i, 0)),
            pl.BlockSpec((bn, 1), lambda i: (i, 0)),
        ],
        out_specs=pl.BlockSpec((bn, d), lambda i: (i, 0)),
        out_shape=jax.ShapeDtypeStruct((n, d), jnp.float32),
    )(h, norm)

    per_w = -(-e // nw)
    k = -(-per_w // _C)
    k += k % 2
    k0 = max(2, (k // 4) * 2)
    k1 = 2 * k - k0
    cap = ns * (k0 + k1) * _C
    rows = ns * _C * (-(-(n + 1) // (ns * _C)))
    packed = jnp.bitwise_or(edge_index[0],
                            jnp.left_shift(edge_index[1], _B))
    pad_val = jnp.int32(n << _B)
    packed = jnp.concatenate(
        [packed, jnp.full((cap - e,), pad_val, jnp.int32)])
    c0 = packed[:ns * k0 * _C].reshape(ns, k0, _C)
    c0 = jnp.concatenate(
        [c0, jnp.full((ns, k1 - k0, _C), pad_val, jnp.int32)], axis=1)
    c1 = packed[ns * k0 * _C:].reshape(ns, k1, _C)
    packed_w = jnp.stack([c0, c1], axis=1).reshape(nw, k1, _C)

    parts = _sc_scatter(normh, packed_w, nw=nw, nc=nc, ks=(k0, k1), rows=rows)

    df2 = W1.shape[0]
    out = pl.pallas_call(
        _main_body,
        grid=(grid,),
        in_specs=[
            pl.BlockSpec((bn, d), lambda i: (i, 0)),
            pl.BlockSpec((2, bn, d), lambda i: (0, i, 0)),
            pl.BlockSpec((bn, 1), lambda i: (i, 0)),
            pl.BlockSpec((d, W1.shape[1]), lambda i: (0, 0)),
            pl.BlockSpec((df2 - d, W1.shape[1]), lambda i: (0, 0)),
            pl.BlockSpec(W2.shape, lambda i: (0, 0)),
        ],
        out_specs=pl.BlockSpec((bn, W2.shape[1]), lambda i: (i, 0)),
        out_shape=jax.ShapeDtypeStruct((n, W2.shape[1]), jnp.float32),
    )(h, parts, norm, W1[:d], W1[d:], W2)
    return out


# speedup vs baseline: 1.1196x; 1.1196x over previous
import functools

import jax
import jax.numpy as jnp
from jax import lax
from jax.experimental import pallas as pl
from jax.experimental.pallas import tpu as pltpu
from jax.experimental.pallas import tpu_sc as plsc

_C = 128
_B = 14


def _prep_body(h_ref, norm_ref, o_ref):
    o_ref[...] = h_ref[...] * norm_ref[...]


def _main_body(h_ref, parts_ref, norm_ref, w1h_ref, w1a_ref, w2_ref, o_ref):
    agg = (parts_ref[0] + parts_ref[1]) * norm_ref[...]
    x = jnp.dot(h_ref[...], w1h_ref[...], preferred_element_type=jnp.float32)
    x = x + jnp.dot(agg, w1a_ref[...], preferred_element_type=jnp.float32)
    x = jnp.maximum(x, 0.0)
    s = jnp.maximum(jnp.sum(x * x, axis=1, keepdims=True), 1e-12)
    x = x * lax.rsqrt(s)
    y = jnp.dot(x, w2_ref[...], preferred_element_type=jnp.float32)
    o_ref[...] = jnp.maximum(y, 0.0)


@functools.partial(jax.jit, static_argnames=("nw", "nc", "k", "rows"))
def _sc_scatter(normh, packed_w, *, nw, nc, k, rows):
    d = normh.shape[1]
    ns = nw // nc
    zch = rows // (ns * _C)
    mask = jnp.int32((1 << _B) - 1)

    @functools.partial(
        pl.kernel,
        out_type=jax.ShapeDtypeStruct((nc, rows, d), jnp.float32),
        mesh=plsc.VectorSubcoreMesh(core_axis_name="c", subcore_axis_name="s"),
        scratch_types=[
            pltpu.VMEM((k, _C), jnp.int32),
            pltpu.VMEM((_C,), jnp.int32),
            pltpu.VMEM((_C,), jnp.int32),
            pltpu.VMEM((_C,), jnp.int32),
            pltpu.VMEM((_C, d), jnp.float32),
            pltpu.VMEM((_C, d), jnp.float32),
            pltpu.VMEM_SHARED((rows, d), jnp.float32),
            pltpu.SemaphoreType.DMA,
            pltpu.SemaphoreType.DMA,
        ],
    )
    def kfn(normh_hbm, packed_hbm, out_hbm, pk_v, s0_v, s1_v, dst_v,
            buf0, buf1, agg, sem0, sem1):
        cid = lax.axis_index("c")
        sid = lax.axis_index("s")
        wid = sid * nc + cid

        pltpu.sync_copy(packed_hbm.at[wid], pk_v)

        zero16 = jnp.zeros((16,), jnp.float32)

        def zrow(r, carry):
            for j in range(d // 16):
                buf0[r, pl.ds(j * 16, 16)] = zero16
            return carry

        lax.fori_loop(0, _C, zrow, 0)
        for j in range(zch):
            pltpu.sync_copy(buf0, agg.at[pl.ds((sid * zch + j) * _C, _C)])
        plsc.subcore_barrier()

        def unpack(row, idx_ref, shift):
            pk_row = pk_v.at[row]
            for j in range(_C // 16):
                p = pk_row[pl.ds(j * 16, 16)]
                if shift:
                    p = lax.shift_right_logical(p, _B)
                idx_ref[pl.ds(j * 16, 16)] = lax.bitwise_and(p, mask)

        def gather(idx_ref, buf, sem):
            return pltpu.async_copy(normh_hbm.at[idx_ref], buf, sem)

        def wait(idx_ref, buf, sem):
            pltpu.make_async_copy(normh_hbm.at[idx_ref], buf, sem).wait()

        unpack(0, s0_v, False)
        gather(s0_v, buf0, sem0)
        unpack(1, s1_v, False)
        gather(s1_v, buf1, sem1)

        def body(i, carry):
            c = 2 * i
            unpack(c, dst_v, True)
            wait(s0_v, buf0, sem0)
            pltpu.sync_copy(buf0, agg.at[dst_v], add=True)

            @pl.when(c + 2 < k)
            def _():
                unpack(c + 2, s0_v, False)
                gather(s0_v, buf0, sem0)

            unpack(c + 1, dst_v, True)
            wait(s1_v, buf1, sem1)
            pltpu.sync_copy(buf1, agg.at[dst_v], add=True)

            @pl.when(c + 3 < k)
            def _():
                unpack(c + 3, s1_v, False)
                gather(s1_v, buf1, sem1)

            return carry

        lax.fori_loop(0, k // 2, body, 0)
        plsc.subcore_barrier()

        for j in range(zch):
            r0 = (sid * zch + j) * _C
            pltpu.sync_copy(agg.at[pl.ds(r0, _C)], out_hbm.at[cid, pl.ds(r0, _C)])

    return kfn(normh, packed_w)


def kernel(h, edge_index, norm, W1, W2):
    n, d = h.shape
    e = edge_index.shape[1]
    info = plsc.get_sparse_core_info()
    nc, ns = info.num_cores, info.num_subcores
    nw = nc * ns

    bn = 1000
    grid = n // bn
    normh = pl.pallas_call(
        _prep_body,
        grid=(grid,),
        in_specs=[
            pl.BlockSpec((bn, d), lambda i: (i, 0)),
            pl.BlockSpec((bn, 1), lambda i: (i, 0)),
        ],
        out_specs=pl.BlockSpec((bn, d), lambda i: (i, 0)),
        out_shape=jax.ShapeDtypeStruct((n, d), jnp.float32),
    )(h, norm)

    per_w = -(-e // nw)
    k = -(-per_w // _C)
    k += k % 2
    cap = nw * k * _C
    rows = ns * _C * (-(-(n + 1) // (ns * _C)))
    packed = jnp.bitwise_or(edge_index[0],
                            jnp.left_shift(edge_index[1], _B))
    pad_val = jnp.int32(n << _B)
    packed = jnp.concatenate(
        [packed, jnp.full((cap - e,), pad_val, jnp.int32)])
    packed_w = packed.reshape(nw, k, _C)

    parts = _sc_scatter(normh, packed_w, nw=nw, nc=nc, k=k, rows=rows)

    df2 = W1.shape[0]
    out = pl.pallas_call(
        _main_body,
        grid=(grid,),
        in_specs=[
            pl.BlockSpec((bn, d), lambda i: (i, 0)),
            pl.BlockSpec((2, bn, d), lambda i: (0, i, 0)),
            pl.BlockSpec((bn, 1), lambda i: (i, 0)),
            pl.BlockSpec((d, W1.shape[1]), lambda i: (0, 0)),
            pl.BlockSpec((df2 - d, W1.shape[1]), lambda i: (0, 0)),
            pl.BlockSpec(W2.shape, lambda i: (0, 0)),
        ],
        out_specs=pl.BlockSpec((bn, W2.shape[1]), lambda i: (i, 0)),
        out_shape=jax.ShapeDtypeStruct((n, W2.shape[1]), jnp.float32),
    )(h, parts, norm, W1[:d], W1[d:], W2)
    return out
